# baseline (speedup 1.0000x reference)
import functools

import jax
import jax.numpy as jnp
from jax import lax
from jax.experimental import pallas as pl
from jax.experimental.pallas import tpu as pltpu
from jax.experimental.pallas import tpu_sc as plsc

N = 10000
D = 128
H = 128
E = 320000

NC, NS, L = 2, 16, 16
DA = 160
CL = 8
CB = 128
NCHUNK = 80
EPT = NCHUNK * CB
E_PAD = NC * NS * EPT
ROWS_PT = 640
N_PAD = NS * ROWS_PT


def _sc_segment_sum(src2, dst2, xa, zeros_big):
    mesh = plsc.VectorSubcoreMesh(core_axis_name="c", subcore_axis_name="s")

    @functools.partial(
        pl.kernel,
        out_type=jax.ShapeDtypeStruct((NC * N_PAD, DA), jnp.bfloat16),
        mesh=mesh,
        compiler_params=pltpu.CompilerParams(use_tc_tiling_on_sc=False),
        scratch_types=[
            pltpu.VMEM((NCHUNK, CB), jnp.int32),
            pltpu.VMEM((NCHUNK, CB), jnp.int32),
            pltpu.VMEM((CB, DA), jnp.bfloat16),
            pltpu.VMEM((CB, DA), jnp.bfloat16),
            pltpu.VMEM_SHARED((N_PAD, DA), jnp.bfloat16),
            pltpu.SemaphoreType.DMA,
            pltpu.SemaphoreType.DMA,
        ],
    )
    def k(src_hbm, dst_hbm, x_hbm, zb_hbm, sum_out,
          src_v, dst_v, rows_v, rows_w, acc_sh, sem_a, sem_b):
        c = lax.axis_index("c")
        s = lax.axis_index("s")
        tid = c * NS + s

        pltpu.sync_copy(src_hbm.at[pl.ds(tid * NCHUNK, NCHUNK)], src_v)
        pltpu.sync_copy(dst_hbm.at[pl.ds(tid * NCHUNK, NCHUNK)], dst_v)

        zoff = s * ROWS_PT
        pltpu.sync_copy(zb_hbm, acc_sh.at[pl.ds(zoff, ROWS_PT)])
        plsc.subcore_barrier()

        pltpu.async_copy(x_hbm.at[src_v.at[0]], rows_v, sem_a)

        def chunk(i, carry):
            ja = 2 * i
            jb = ja + 1
            pltpu.async_copy(x_hbm.at[src_v.at[jb]], rows_w, sem_b)
            pltpu.make_async_copy(x_hbm.at[src_v.at[ja]], rows_v, sem_a).wait()
            pltpu.sync_copy(rows_v, acc_sh.at[dst_v.at[ja]], add=True)

            @pl.when(jb + 1 < NCHUNK)
            def _():
                pltpu.async_copy(x_hbm.at[src_v.at[jb + 1]], rows_v, sem_a)

            pltpu.make_async_copy(x_hbm.at[src_v.at[jb]], rows_w, sem_b).wait()
            pltpu.sync_copy(rows_w, acc_sh.at[dst_v.at[jb]], add=True)
            return carry

        lax.fori_loop(0, NCHUNK // 2, chunk, None)
        plsc.subcore_barrier()

        pltpu.sync_copy(acc_sh.at[pl.ds(zoff, ROWS_PT)],
                        sum_out.at[pl.ds(c * N_PAD + zoff, ROWS_PT)])

    return k(src2, dst2, xa, zeros_big)


def _tc_finish(sum0, sum1, cnt0, cnt1, x, W_l, b_l, W_r):
    BLK = 1000
    dn = (((1,), (1,)), ((), ()))

    def body(s0, s1, c0, c1, xr, wl, bl, wr, out):
        ssum = s0[...].astype(jnp.float32) + s1[...].astype(jnp.float32)
        cnt_col = (c0[:, 0:1].astype(jnp.float32)
                   + c1[:, 0:1].astype(jnp.float32))
        mean = ssum / jnp.maximum(cnt_col, 1.0)
        h = (lax.dot_general(mean, wl[...], dn,
                             precision=lax.Precision.HIGHEST,
                             preferred_element_type=jnp.float32)
             + bl[...]
             + lax.dot_general(xr[...], wr[...], dn,
                               precision=lax.Precision.HIGHEST,
                               preferred_element_type=jnp.float32))
        nrm = jnp.sqrt(jnp.sum(h * h, axis=1, keepdims=True))
        out[...] = h / jnp.maximum(nrm, 1e-12)

    return pl.pallas_call(
        body,
        grid=(N // BLK,),
        in_specs=[
            pl.BlockSpec((BLK, D), lambda i: (i, 0)),
            pl.BlockSpec((BLK, D), lambda i: (i, 0)),
            pl.BlockSpec((BLK, CL), lambda i: (i, 0)),
            pl.BlockSpec((BLK, CL), lambda i: (i, 0)),
            pl.BlockSpec((BLK, D), lambda i: (i, 0)),
            pl.BlockSpec((H, D), lambda i: (0, 0)),
            pl.BlockSpec((1, H), lambda i: (0, 0)),
            pl.BlockSpec((H, D), lambda i: (0, 0)),
        ],
        out_specs=pl.BlockSpec((BLK, H), lambda i: (i, 0)),
        out_shape=jax.ShapeDtypeStruct((N, H), jnp.float32),
    )(sum0, sum1, cnt0, cnt1, x, W_l, b_l.reshape(1, H), W_r)


def kernel(edge_index, x, W_l, b_l, W_r):
    src = edge_index[0]
    dst = edge_index[1]
    pad = E_PAD - E
    src_p = jnp.concatenate(
        [src, jnp.zeros((pad,), jnp.int32)]).reshape(NC * NS * NCHUNK, CB)
    dst_p = jnp.concatenate(
        [dst, jnp.full((pad,), N_PAD - 1, jnp.int32)]).reshape(NC * NS * NCHUNK, CB)
    xa = jnp.concatenate(
        [x.astype(jnp.bfloat16),
         jnp.ones((N, 1), jnp.bfloat16),
         jnp.zeros((N, DA - D - 1), jnp.bfloat16)], axis=1)
    zeros_big = jnp.zeros((ROWS_PT, DA), jnp.bfloat16)
    sums = _sc_segment_sum(src_p, dst_p, xa, zeros_big)
    sum0, sum1 = sums[:N, :D], sums[N_PAD:N_PAD + N, :D]
    cnt0 = sums[:N, D:D + CL]
    cnt1 = sums[N_PAD:N_PAD + N, D:D + CL]
    return _tc_finish(sum0, sum1, cnt0, cnt1, x, W_l, b_l, W_r)


# speedup vs baseline: 2.2330x; 2.2330x over previous
import functools

import jax
import jax.numpy as jnp
from jax import lax
from jax.experimental import pallas as pl
from jax.experimental.pallas import tpu as pltpu
from jax.experimental.pallas import tpu_sc as plsc

N = 10000
D = 128
H = 128
E = 320000

NC, NS, L = 2, 16, 16
DH = D // 2
CL = 8
CB = 128
NCHUNK = 160
EPT = NCHUNK * CB
E_PAD = NS * EPT
ROWS_PT = 640
N_PAD = NS * ROWS_PT


def _sc_segment_sum(src2, dst2, xb2, zeros_big, zeros_small, ones_small):
    mesh = plsc.VectorSubcoreMesh(core_axis_name="c", subcore_axis_name="s")

    @functools.partial(
        pl.kernel,
        out_type=(
            jax.ShapeDtypeStruct((NC * N_PAD, DH), jnp.bfloat16),
            jax.ShapeDtypeStruct((NC * N_PAD, CL), jnp.float32),
        ),
        mesh=mesh,
        compiler_params=pltpu.CompilerParams(use_tc_tiling_on_sc=False),
        scratch_types=[
            pltpu.VMEM((NCHUNK, CB), jnp.int32),
            pltpu.VMEM((NCHUNK, CB), jnp.int32),
            pltpu.VMEM((CB, DH), jnp.bfloat16),
            pltpu.VMEM((CB, DH), jnp.bfloat16),
            pltpu.VMEM((CB, CL), jnp.float32),
            pltpu.VMEM_SHARED((N_PAD, DH), jnp.bfloat16),
            pltpu.VMEM_SHARED((N_PAD, DH), jnp.bfloat16),
            pltpu.VMEM_SHARED((N_PAD, CL), jnp.float32),
            pltpu.SemaphoreType.DMA,
            pltpu.SemaphoreType.DMA,
        ],
    )
    def k(src_hbm, dst_hbm, x_hbm, zb_hbm, zs_hbm, ones_hbm, sum_out, cnt_out,
          src_v, dst_v, rows_v, rows_w, col_v, x_sh, acc_sh, cnt_sh,
          sem_a, sem_b):
        c = lax.axis_index("c")
        s = lax.axis_index("s")

        pltpu.sync_copy(src_hbm.at[pl.ds(s * NCHUNK, NCHUNK)], src_v)
        pltpu.sync_copy(dst_hbm.at[pl.ds(s * NCHUNK, NCHUNK)], dst_v)

        zoff = s * ROWS_PT
        pltpu.sync_copy(x_hbm.at[pl.ds(c * N_PAD + zoff, ROWS_PT)],
                        x_sh.at[pl.ds(zoff, ROWS_PT)])
        pltpu.sync_copy(zb_hbm, acc_sh.at[pl.ds(zoff, ROWS_PT)])
        pltpu.sync_copy(zs_hbm, cnt_sh.at[pl.ds(zoff, ROWS_PT)])
        pltpu.sync_copy(ones_hbm, col_v)
        plsc.subcore_barrier()

        pltpu.async_copy(x_sh.at[src_v.at[0]], rows_v, sem_a)

        def chunk(i, carry):
            ja = 2 * i
            jb = ja + 1
            pltpu.async_copy(x_sh.at[src_v.at[jb]], rows_w, sem_b)
            pltpu.make_async_copy(x_sh.at[src_v.at[ja]], rows_v, sem_a).wait()
            pltpu.sync_copy(rows_v, acc_sh.at[dst_v.at[ja]], add=True)

            @pl.when(c == 0)
            def _():
                pltpu.sync_copy(col_v, cnt_sh.at[dst_v.at[ja]], add=True)

            @pl.when(jb + 1 < NCHUNK)
            def _():
                pltpu.async_copy(x_sh.at[src_v.at[jb + 1]], rows_v, sem_a)

            pltpu.make_async_copy(x_sh.at[src_v.at[jb]], rows_w, sem_b).wait()
            pltpu.sync_copy(rows_w, acc_sh.at[dst_v.at[jb]], add=True)

            @pl.when(c == 1)
            def _():
                pltpu.sync_copy(col_v, cnt_sh.at[dst_v.at[jb]], add=True)

            return carry

        lax.fori_loop(0, NCHUNK // 2, chunk, None)
        plsc.subcore_barrier()

        pltpu.sync_copy(acc_sh.at[pl.ds(zoff, ROWS_PT)],
                        sum_out.at[pl.ds(c * N_PAD + zoff, ROWS_PT)])
        pltpu.sync_copy(cnt_sh.at[pl.ds(zoff, ROWS_PT)],
                        cnt_out.at[pl.ds(c * N_PAD + zoff, ROWS_PT)])

    return k(src2, dst2, xb2, zeros_big, zeros_small, ones_small)


def _tc_finish(sum_lo, sum_hi, cnt0, cnt1, x, W_l, b_l, W_r):
    BLK = 1000
    dn = (((1,), (1,)), ((), ()))

    def body(slo, shi, c0, c1, xr, wl, bl, wr, out):
        ssum = jnp.concatenate(
            [slo[...].astype(jnp.float32), shi[...].astype(jnp.float32)],
            axis=1)
        cnt_col = c0[:, 0:1] + c1[:, 0:1]
        mean = ssum / jnp.maximum(cnt_col, 1.0)
        h = (lax.dot_general(mean, wl[...], dn,
                             precision=lax.Precision.HIGHEST,
                             preferred_element_type=jnp.float32)
             + bl[...]
             + lax.dot_general(xr[...], wr[...], dn,
                               precision=lax.Precision.HIGHEST,
                               preferred_element_type=jnp.float32))
        nrm = jnp.sqrt(jnp.sum(h * h, axis=1, keepdims=True))
        out[...] = h / jnp.maximum(nrm, 1e-12)

    return pl.pallas_call(
        body,
        grid=(N // BLK,),
        in_specs=[
            pl.BlockSpec((BLK, DH), lambda i: (i, 0)),
            pl.BlockSpec((BLK, DH), lambda i: (i, 0)),
            pl.BlockSpec((BLK, CL), lambda i: (i, 0)),
            pl.BlockSpec((BLK, CL), lambda i: (i, 0)),
            pl.BlockSpec((BLK, D), lambda i: (i, 0)),
            pl.BlockSpec((H, D), lambda i: (0, 0)),
            pl.BlockSpec((1, H), lambda i: (0, 0)),
            pl.BlockSpec((H, D), lambda i: (0, 0)),
        ],
        out_specs=pl.BlockSpec((BLK, H), lambda i: (i, 0)),
        out_shape=jax.ShapeDtypeStruct((N, H), jnp.float32),
    )(sum_lo, sum_hi, cnt0, cnt1, x, W_l, b_l.reshape(1, H), W_r)


def kernel(edge_index, x, W_l, b_l, W_r):
    src = edge_index[0]
    dst = edge_index[1]
    pad = E_PAD - E
    src_p = jnp.concatenate(
        [src, jnp.zeros((pad,), jnp.int32)]).reshape(NS * NCHUNK, CB)
    dst_p = jnp.concatenate(
        [dst, jnp.full((pad,), N_PAD - 1, jnp.int32)]).reshape(NS * NCHUNK, CB)
    xb = x.astype(jnp.bfloat16)
    xbp = jnp.concatenate([xb, jnp.zeros((N_PAD - N, D), jnp.bfloat16)])
    xb2 = jnp.concatenate([xbp[:, :DH], xbp[:, DH:]])
    zeros_big = jnp.zeros((ROWS_PT, DH), jnp.bfloat16)
    zeros_small = jnp.zeros((ROWS_PT, CL), jnp.float32)
    ones_small = jnp.ones((CB, CL), jnp.float32)
    sums, cnts = _sc_segment_sum(src_p, dst_p, xb2, zeros_big, zeros_small,
                                 ones_small)
    sum_lo, sum_hi = sums[:N], sums[N_PAD:N_PAD + N]
    cnt0, cnt1 = cnts[:N], cnts[N_PAD:N_PAD + N]
    return _tc_finish(sum_lo, sum_hi, cnt0, cnt1, x, W_l, b_l, W_r)
